# Initial kernel scaffold; baseline (speedup 1.0000x reference)
import jax
import jax.numpy as jnp
from jax.experimental import pallas as pl


def kernel(coords, normals):
    raise NotImplementedError("write your pallas kernel here")


# speedup vs baseline: 27.9405x; 27.9405x over previous
import jax
import jax.numpy as jnp
from jax.experimental import pallas as pl
from jax.experimental.pallas import tpu as pltpu

_B = 2
_N = 4096
_K = 32
_R2 = 0.15 * 0.15
_RT = 256
_CT = 512


def _ppf_body(xc_ref, xr_ref, nr_ref, t_ref, o_ref, cnt_ref, mn_ref):
    ct = pl.program_id(2)
    nct = pl.num_programs(2)

    @pl.when(ct == 0)
    def _init():
        cnt_ref[...] = jnp.zeros_like(cnt_ref)
        mn_ref[...] = jnp.full_like(mn_ref, 2.0)

    xr = xr_ref[0]
    nr = nr_ref[0]
    xc = xc_ref[0]
    xi, yi, zi = xr[:, 0:1], xr[:, 1:2], xr[:, 2:3]
    nx, ny, nz = nr[:, 0:1], nr[:, 1:2], nr[:, 2:3]
    xj, yj, zj = xc[0:1, :], xc[1:2, :], xc[2:3, :]

    sqi = xi * xi + yi * yi + zi * zi
    sqj = xj * xj + yj * yj + zj * zj
    e = jax.lax.dot_general(
        xr, xc, (((1,), (0,)), ((), ())),
        preferred_element_type=jnp.float32)
    d2 = jnp.maximum(sqi + sqj - 2.0 * e, 0.0)
    m = d2 <= _R2

    dx = xj - xi
    dy = yj - yi
    dz = zj - zi
    dot = nx * dx + ny * dy + nz * dz
    cx = ny * dz - nz * dy
    cy = nz * dx - nx * dz
    cz = nx * dy - ny * dx
    sq = cx * cx + cy * cy + cz * cz
    adot = jnp.abs(dot)
    f = dot * adot / (dot * dot + sq)
    f = jnp.where(sq < 1e-20, jnp.where(dot <= -1e-10, -1.0, 1.0), f)

    mf = m.astype(jnp.bfloat16)
    excl = jax.lax.dot_general(
        mf, t_ref[...], (((1,), (0,)), ((), ())),
        preferred_element_type=jnp.float32)
    cnt = cnt_ref[...]
    inc = m & ((cnt + excl) < float(_K))
    fm = jnp.where(inc, f, 2.0)
    mn_ref[...] = jnp.minimum(mn_ref[...], jnp.min(fm, axis=1, keepdims=True))
    cnt_ref[...] = cnt + jnp.sum(m.astype(jnp.float32), axis=1, keepdims=True)

    @pl.when(ct == nct - 1)
    def _fin():
        fmin = mn_ref[...]
        s = jnp.sqrt(jnp.abs(fmin))
        c = jnp.clip(jnp.where(fmin >= 0.0, s, -s), -1.0, 1.0)
        sn = jnp.sqrt(jnp.maximum(1.0 - c * c, 0.0))
        o_ref[0] = jnp.arctan2(sn, c)


def _build(interpret=False):
    grid = (_B, _N // _RT, _N // _CT)
    return pl.pallas_call(
        _ppf_body,
        grid=grid,
        in_specs=[
            pl.BlockSpec((1, 3, _CT), lambda b, r, c: (b, 0, c)),
            pl.BlockSpec((1, _RT, 3), lambda b, r, c: (b, r, 0)),
            pl.BlockSpec((1, _RT, 3), lambda b, r, c: (b, r, 0)),
            pl.BlockSpec((_CT, _CT), lambda b, r, c: (0, 0)),
        ],
        out_specs=pl.BlockSpec((1, _RT, 1), lambda b, r, c: (b, r, 0)),
        out_shape=jax.ShapeDtypeStruct((_B, _N, 1), jnp.float32),
        scratch_shapes=[
            pltpu.VMEM((_RT, 1), jnp.float32),
            pltpu.VMEM((_RT, 1), jnp.float32),
        ],
        compiler_params=pltpu.CompilerParams(
            dimension_semantics=("arbitrary", "arbitrary", "arbitrary")),
        interpret=interpret,
    )


def kernel(coords, normals):
    xyz_t = jnp.transpose(coords, (0, 2, 1))
    nrm_t = jnp.transpose(normals, (0, 2, 1))
    a = jnp.arange(_CT, dtype=jnp.int32)
    tmat = (a[:, None] < a[None, :]).astype(jnp.bfloat16)
    out = _build()(coords, xyz_t, nrm_t, tmat)
    return out.reshape(_B, 1, _N)
